# Initial kernel scaffold; baseline (speedup 1.0000x reference)
import jax
import jax.numpy as jnp
from jax.experimental import pallas as pl


def kernel(x, enc_w1, enc_b1, enc_w2, enc_b2, codebooks, dec_w1, dec_b1, dec_w2, dec_b2):
    raise NotImplementedError("write your pallas kernel here")


# speedup vs baseline: 1.0893x; 1.0893x over previous
import functools

import jax
import jax.numpy as jnp
from jax import lax
from jax.experimental import pallas as pl

B, S = 8, 576
N = B * S
INPUT_DIM = 768
BOTTLENECK = 256
NUM_Q = 8
K = 1024
CW = 0.25

TILE = 512
GRID = N // TILE


def _gelu(x):
    return 0.5 * x * (1.0 + lax.erf(x * (0.5 ** 0.5)))


def _fused_kernel(x_ref, w1_ref, b1_ref, w2_ref, b2_ref, cb_ref,
                  dw1_ref, db1_ref, dw2_ref, db2_ref,
                  z_ref, qst_ref, idx_ref, rec_ref, closs_ref):
    i = pl.program_id(0)

    x = x_ref[...]
    h = _gelu(jnp.dot(x, w1_ref[...], preferred_element_type=jnp.float32)
              + b1_ref[...])
    z = jnp.dot(h, w2_ref[...], preferred_element_type=jnp.float32) + b2_ref[...]

    residual = z
    quant = jnp.zeros_like(z)
    closs = jnp.float32(0.0)
    for q in range(NUM_Q):
        cb = cb_ref[q]
        r2 = jnp.sum(residual ** 2, axis=-1, keepdims=True)
        c2 = jnp.sum(cb ** 2, axis=-1)
        dots = lax.dot_general(residual, cb, (((1,), (1,)), ((), ())),
                               preferred_element_type=jnp.float32)
        dists = r2 - 2.0 * dots + c2
        idx = jnp.argmin(dists, axis=-1)
        onehot = (lax.broadcasted_iota(jnp.int32, (TILE, K), 1)
                  == idx[:, None]).astype(jnp.float32)
        qv = lax.dot_general(onehot, cb, (((1,), (0,)), ((), ())),
                             preferred_element_type=jnp.float32,
                             precision=lax.Precision.HIGHEST)
        closs = closs + jnp.sum((residual - qv) ** 2)
        idx_ref[q, :] = idx
        quant = quant + qv
        residual = residual - qv

    qst = z + (quant - z)
    h2 = _gelu(jnp.dot(qst, dw1_ref[...], preferred_element_type=jnp.float32)
               + db1_ref[...])
    rec = jnp.dot(h2, dw2_ref[...], preferred_element_type=jnp.float32) + db2_ref[...]

    z_ref[...] = z
    qst_ref[...] = qst
    rec_ref[...] = rec

    @pl.when(i == 0)
    def _init():
        closs_ref[...] = jnp.zeros_like(closs_ref)

    closs_ref[...] += jnp.reshape(closs, (1, 1))


@functools.partial(jax.jit, static_argnames=())
def kernel(x, enc_w1, enc_b1, enc_w2, enc_b2, codebooks,
           dec_w1, dec_b1, dec_w2, dec_b2):
    xr = x.reshape(N, INPUT_DIM)
    b1 = enc_b1.reshape(1, INPUT_DIM)
    b2 = enc_b2.reshape(1, BOTTLENECK)
    db1 = dec_b1.reshape(1, INPUT_DIM)
    db2 = dec_b2.reshape(1, INPUT_DIM)

    full = lambda shape: pl.BlockSpec(shape, lambda i: (0,) * len(shape))
    z, qst, idx, rec, closs = pl.pallas_call(
        _fused_kernel,
        grid=(GRID,),
        in_specs=[
            pl.BlockSpec((TILE, INPUT_DIM), lambda i: (i, 0)),
            full((INPUT_DIM, INPUT_DIM)),
            full((1, INPUT_DIM)),
            full((INPUT_DIM, BOTTLENECK)),
            full((1, BOTTLENECK)),
            full((NUM_Q, K, BOTTLENECK)),
            full((BOTTLENECK, INPUT_DIM)),
            full((1, INPUT_DIM)),
            full((INPUT_DIM, INPUT_DIM)),
            full((1, INPUT_DIM)),
        ],
        out_specs=[
            pl.BlockSpec((TILE, BOTTLENECK), lambda i: (i, 0)),
            pl.BlockSpec((TILE, BOTTLENECK), lambda i: (i, 0)),
            pl.BlockSpec((NUM_Q, TILE), lambda i: (0, i)),
            pl.BlockSpec((TILE, INPUT_DIM), lambda i: (i, 0)),
            pl.BlockSpec((1, 1), lambda i: (0, 0)),
        ],
        out_shape=[
            jax.ShapeDtypeStruct((N, BOTTLENECK), jnp.float32),
            jax.ShapeDtypeStruct((N, BOTTLENECK), jnp.float32),
            jax.ShapeDtypeStruct((NUM_Q, N), jnp.int32),
            jax.ShapeDtypeStruct((N, INPUT_DIM), jnp.float32),
            jax.ShapeDtypeStruct((1, 1), jnp.float32),
        ],
    )(xr, enc_w1, b1, enc_w2, b2, codebooks, dec_w1, db1, dec_w2, db2)

    commitment_loss = closs[0, 0] * (CW / (N * BOTTLENECK))
    return (z.reshape(B, S, BOTTLENECK),
            qst.reshape(B, S, BOTTLENECK),
            idx.reshape(NUM_Q, B, S),
            rec.reshape(B, S, INPUT_DIM),
            commitment_loss)


# speedup vs baseline: 1.6036x; 1.4722x over previous
import functools

import jax
import jax.numpy as jnp
from jax import lax
from jax.experimental import pallas as pl

B, S = 8, 576
N = B * S
INPUT_DIM = 768
BOTTLENECK = 256
NUM_Q = 8
K = 1024
CW = 0.25

TILE = 512
GRID = N // TILE


def _gelu(x):
    return 0.5 * x * (1.0 + lax.erf(x * (0.5 ** 0.5)))


def _fused_kernel(x_ref, w1_ref, b1_ref, w2_ref, b2_ref, cb_ref,
                  dw1_ref, db1_ref, dw2_ref, db2_ref,
                  z_ref, qst_ref, idx_ref, rec_ref, closs_ref):
    i = pl.program_id(0)

    x = x_ref[...]
    h = _gelu(jnp.dot(x, w1_ref[...], preferred_element_type=jnp.float32)
              + b1_ref[...])
    z = jnp.dot(h, w2_ref[...], preferred_element_type=jnp.float32) + b2_ref[...]

    residual = z
    quant = jnp.zeros_like(z)
    closs = jnp.float32(0.0)
    for q in range(NUM_Q):
        cb = cb_ref[q]
        r2 = jnp.sum(residual ** 2, axis=-1, keepdims=True)
        c2 = jnp.sum(cb ** 2, axis=-1)
        dots = lax.dot_general(residual, cb, (((1,), (1,)), ((), ())),
                               preferred_element_type=jnp.float32)
        dists = r2 - 2.0 * dots + c2
        idx = jnp.argmin(dists, axis=-1)
        onehot = (lax.broadcasted_iota(jnp.int32, (TILE, K), 1)
                  == idx[:, None]).astype(jnp.bfloat16)
        cb_hi = cb.astype(jnp.bfloat16)
        rem1 = cb - cb_hi.astype(jnp.float32)
        cb_mid = rem1.astype(jnp.bfloat16)
        cb_lo = (rem1 - cb_mid.astype(jnp.float32)).astype(jnp.bfloat16)
        dn = (((1,), (0,)), ((), ()))
        qv_hi = lax.dot_general(onehot, cb_hi, dn,
                                preferred_element_type=jnp.float32)
        qv_mid = lax.dot_general(onehot, cb_mid, dn,
                                 preferred_element_type=jnp.float32)
        qv_lo = lax.dot_general(onehot, cb_lo, dn,
                                preferred_element_type=jnp.float32)
        qv = (qv_hi + qv_mid) + qv_lo
        closs = closs + jnp.sum((residual - qv) ** 2)
        idx_ref[q, :] = idx
        quant = quant + qv
        residual = residual - qv

    qst = z + (quant - z)
    h2 = _gelu(jnp.dot(qst, dw1_ref[...], preferred_element_type=jnp.float32)
               + db1_ref[...])
    rec = jnp.dot(h2, dw2_ref[...], preferred_element_type=jnp.float32) + db2_ref[...]

    z_ref[...] = z
    qst_ref[...] = qst
    rec_ref[...] = rec

    @pl.when(i == 0)
    def _init():
        closs_ref[...] = jnp.zeros_like(closs_ref)

    closs_ref[...] += jnp.reshape(closs, (1, 1))


@functools.partial(jax.jit, static_argnames=())
def kernel(x, enc_w1, enc_b1, enc_w2, enc_b2, codebooks,
           dec_w1, dec_b1, dec_w2, dec_b2):
    xr = x.reshape(N, INPUT_DIM)
    b1 = enc_b1.reshape(1, INPUT_DIM)
    b2 = enc_b2.reshape(1, BOTTLENECK)
    db1 = dec_b1.reshape(1, INPUT_DIM)
    db2 = dec_b2.reshape(1, INPUT_DIM)

    full = lambda shape: pl.BlockSpec(shape, lambda i: (0,) * len(shape))
    z, qst, idx, rec, closs = pl.pallas_call(
        _fused_kernel,
        grid=(GRID,),
        in_specs=[
            pl.BlockSpec((TILE, INPUT_DIM), lambda i: (i, 0)),
            full((INPUT_DIM, INPUT_DIM)),
            full((1, INPUT_DIM)),
            full((INPUT_DIM, BOTTLENECK)),
            full((1, BOTTLENECK)),
            full((NUM_Q, K, BOTTLENECK)),
            full((BOTTLENECK, INPUT_DIM)),
            full((1, INPUT_DIM)),
            full((INPUT_DIM, INPUT_DIM)),
            full((1, INPUT_DIM)),
        ],
        out_specs=[
            pl.BlockSpec((TILE, BOTTLENECK), lambda i: (i, 0)),
            pl.BlockSpec((TILE, BOTTLENECK), lambda i: (i, 0)),
            pl.BlockSpec((NUM_Q, TILE), lambda i: (0, i)),
            pl.BlockSpec((TILE, INPUT_DIM), lambda i: (i, 0)),
            pl.BlockSpec((1, 1), lambda i: (0, 0)),
        ],
        out_shape=[
            jax.ShapeDtypeStruct((N, BOTTLENECK), jnp.float32),
            jax.ShapeDtypeStruct((N, BOTTLENECK), jnp.float32),
            jax.ShapeDtypeStruct((NUM_Q, N), jnp.int32),
            jax.ShapeDtypeStruct((N, INPUT_DIM), jnp.float32),
            jax.ShapeDtypeStruct((1, 1), jnp.float32),
        ],
    )(xr, enc_w1, b1, enc_w2, b2, codebooks, dec_w1, db1, dec_w2, db2)

    commitment_loss = closs[0, 0] * (CW / (N * BOTTLENECK))
    return (z.reshape(B, S, BOTTLENECK),
            qst.reshape(B, S, BOTTLENECK),
            idx.reshape(NUM_Q, B, S),
            rec.reshape(B, S, INPUT_DIM),
            commitment_loss)
